# baseline (speedup 1.0000x reference)
import functools
import jax
import jax.numpy as jnp
from jax import lax
from jax.experimental import pallas as pl
from jax.experimental.pallas import tpu as pltpu
from jax.experimental.pallas import tpu_sc as plsc

_NC, _NS, _L = 2, 16, 16
_NW = _NC * _NS

_TCB = 256
_SCB = 8
_NBUF = 2

_TC_32 = 16


def _tc_add(x, w, row0, nrows, T, D):
    def body(x_hbm, w_vmem, o_hbm, ib0, ib1, ob0, ob1, is0, is1, os0, os1):
        ibufs = (ib0, ib1)
        obufs = (ob0, ob1)
        isems = (is0, is1)
        osems = (os0, os1)
        nchunk = nrows // _TCB
        wv = w_vmem[...]

        def in_copy(g, s):
            return pltpu.make_async_copy(
                x_hbm.at[pl.ds(row0 + g * _TCB, _TCB)], ibufs[s], isems[s])

        def out_copy(g, s):
            return pltpu.make_async_copy(
                obufs[s], o_hbm.at[pl.ds(g * _TCB, _TCB)], osems[s])

        for s in range(_NBUF):
            in_copy(s, s).start()

        def pair(gg, carry):
            for s in range(_NBUF):
                g = gg * _NBUF + s
                in_copy(g, s).wait()

                @pl.when(g >= _NBUF)
                def _():
                    out_copy(g - _NBUF, s).wait()

                obufs[s][...] = ibufs[s][...] + wv
                out_copy(g, s).start()

                @pl.when(g + _NBUF < nchunk)
                def _():
                    in_copy(g + _NBUF, s).start()
            return carry

        lax.fori_loop(0, nchunk // _NBUF, pair, 0)
        for s in range(_NBUF):
            out_copy(nchunk - _NBUF + s, s).wait()

    return pl.pallas_call(
        body,
        in_specs=[
            pl.BlockSpec(memory_space=pl.ANY),
            pl.BlockSpec(memory_space=pltpu.VMEM),
        ],
        out_specs=pl.BlockSpec(memory_space=pl.ANY),
        out_shape=jax.ShapeDtypeStruct((nrows, T, D), jnp.float32),
        scratch_shapes=[
            pltpu.VMEM((_TCB, T, D), jnp.float32),
            pltpu.VMEM((_TCB, T, D), jnp.float32),
            pltpu.VMEM((_TCB, T, D), jnp.float32),
            pltpu.VMEM((_TCB, T, D), jnp.float32),
            pltpu.SemaphoreType.DMA,
            pltpu.SemaphoreType.DMA,
            pltpu.SemaphoreType.DMA,
            pltpu.SemaphoreType.DMA,
        ],
    )(x, w[None])


def _sc_add(x, w, row0, nrows, T, D):
    rows_w = nrows // _NW
    nchunk = rows_w // _SCB
    mesh = plsc.VectorSubcoreMesh(core_axis_name="c", subcore_axis_name="s")

    @functools.partial(
        pl.kernel,
        out_type=jax.ShapeDtypeStruct((nrows, T, D), jnp.float32),
        mesh=mesh,
        scratch_types=[
            pltpu.VMEM((_SCB, T, D), jnp.float32),
            pltpu.VMEM((_SCB, T, D), jnp.float32),
            pltpu.VMEM((_SCB, T, D), jnp.float32),
            pltpu.VMEM((_SCB, T, D), jnp.float32),
            pltpu.VMEM((T, D), jnp.float32),
            pltpu.SemaphoreType.DMA,
            pltpu.SemaphoreType.DMA,
            pltpu.SemaphoreType.DMA,
            pltpu.SemaphoreType.DMA,
        ],
    )
    def k(x_hbm, w_hbm, o_hbm, ibuf0, ibuf1, obuf0, obuf1, w_v,
          isem0, isem1, osem0, osem1):
        ibufs = (ibuf0, ibuf1)
        obufs = (obuf0, obuf1)
        isems = (isem0, isem1)
        osems = (osem0, osem1)
        wid = lax.axis_index("s") * _NC + lax.axis_index("c")
        in_base = row0 + wid * rows_w
        out_base = wid * rows_w

        pltpu.sync_copy(w_hbm, w_v)

        def in_copy(g, s):
            return pltpu.make_async_copy(
                x_hbm.at[pl.ds(in_base + g * _SCB, _SCB)], ibufs[s], isems[s])

        def out_copy(g, s):
            return pltpu.make_async_copy(
                obufs[s], o_hbm.at[pl.ds(out_base + g * _SCB, _SCB)], osems[s])

        for s in range(_NBUF):
            in_copy(s, s).start()

        def add_chunk(src, dst):
            def t_body(t, c):
                for j in range(D // _L):
                    sl = pl.ds(j * _L, _L)
                    wv = w_v[t, sl]
                    for i in range(_SCB):
                        dst[i, t, sl] = src[i, t, sl] + wv
                return c
            lax.fori_loop(0, T, t_body, 0)

        def pair(gg, carry):
            for s in range(_NBUF):
                g = gg * _NBUF + s
                in_copy(g, s).wait()

                @pl.when(g >= _NBUF)
                def _():
                    out_copy(g - _NBUF, s).wait()

                add_chunk(ibufs[s], obufs[s])
                out_copy(g, s).start()

                @pl.when(g + _NBUF < nchunk)
                def _():
                    in_copy(g + _NBUF, s).start()
            return carry

        lax.fori_loop(0, nchunk // _NBUF, pair, 0)
        for s in range(_NBUF):
            out_copy(nchunk - _NBUF + s, s).wait()

    return k(x, w)


def kernel(x, encoding_weight):
    B, T, D = x.shape
    tc_rows = (B * _TC_32) // 32
    lo = _tc_add(x, encoding_weight, 0, tc_rows, T, D)
    hi = _sc_add(x, encoding_weight, tc_rows, B - tc_rows, T, D)
    return jnp.concatenate([lo, hi], axis=0)


# speedup vs baseline: 1.2831x; 1.2831x over previous
import functools
import jax
import jax.numpy as jnp
from jax import lax
from jax.experimental import pallas as pl
from jax.experimental.pallas import tpu as pltpu
from jax.experimental.pallas import tpu_sc as plsc

_NC, _NS, _L = 2, 16, 16
_NW = _NC * _NS
_CB = 4
_NBUF = 4


def _make(B, T, D):
    rows_w = B // _NW
    nchunk = rows_w // _CB
    assert nchunk % _NBUF == 0
    mesh = plsc.VectorSubcoreMesh(core_axis_name="c", subcore_axis_name="s")

    @functools.partial(
        pl.kernel,
        out_type=jax.ShapeDtypeStruct((B, T, D), jnp.float32),
        mesh=mesh,
        scratch_types=(
            [pltpu.VMEM((_CB, T, D), jnp.float32)] * (2 * _NBUF)
            + [pltpu.VMEM((T, D), jnp.float32)]
            + [pltpu.SemaphoreType.DMA] * (2 * _NBUF)
        ),
    )
    def k(x_hbm, w_hbm, o_hbm, *rest):
        ibufs = rest[:_NBUF]
        obufs = rest[_NBUF:2 * _NBUF]
        w_v = rest[2 * _NBUF]
        isems = rest[2 * _NBUF + 1:2 * _NBUF + 1 + _NBUF]
        osems = rest[2 * _NBUF + 1 + _NBUF:]
        wid = lax.axis_index("s") * _NC + lax.axis_index("c")
        base = wid * rows_w

        pltpu.sync_copy(w_hbm, w_v)

        def in_copy(g, s):
            return pltpu.make_async_copy(
                x_hbm.at[pl.ds(base + g * _CB, _CB)], ibufs[s], isems[s])

        def out_copy(g, s):
            return pltpu.make_async_copy(
                obufs[s], o_hbm.at[pl.ds(base + g * _CB, _CB)], osems[s])

        for s in range(_NBUF):
            in_copy(s, s).start()

        def add_chunk(src, dst):
            def t_body(t, c):
                for j in range(D // _L):
                    sl = pl.ds(j * _L, _L)
                    wv = w_v[t, sl]
                    for i in range(_CB):
                        dst[i, t, sl] = src[i, t, sl] + wv
                return c
            lax.fori_loop(0, T, t_body, 0)

        def pair(gg, carry):
            for s in range(_NBUF):
                g = gg * _NBUF + s
                in_copy(g, s).wait()

                @pl.when(g >= _NBUF)
                def _():
                    out_copy(g - _NBUF, s).wait()

                add_chunk(ibufs[s], obufs[s])
                out_copy(g, s).start()
                nxt = g + _NBUF

                @pl.when(nxt < nchunk)
                def _():
                    in_copy(nxt, s).start()
            return carry

        lax.fori_loop(0, nchunk // _NBUF, pair, 0)

        for s in range(_NBUF):
            out_copy(nchunk - _NBUF + s, s).wait()

    return k


def kernel(x, encoding_weight):
    B, T, D = x.shape
    return _make(B, T, D)(x, encoding_weight)


# speedup vs baseline: 1.2852x; 1.0016x over previous
import functools
import jax
import jax.numpy as jnp
from jax import lax
from jax.experimental import pallas as pl
from jax.experimental.pallas import tpu as pltpu
from jax.experimental.pallas import tpu_sc as plsc

_NC, _NS, _L = 2, 16, 16
_NW = _NC * _NS
_CB = 4
_NBUF = 4


def _make(B, T, D):
    rows_w = B // _NW
    nchunk = rows_w // _CB
    assert nchunk % _NBUF == 0
    mesh = plsc.VectorSubcoreMesh(core_axis_name="c", subcore_axis_name="s")

    @functools.partial(
        pl.kernel,
        out_type=jax.ShapeDtypeStruct((B, T, D), jnp.float32),
        mesh=mesh,
        scratch_types=(
            [pltpu.VMEM((_CB, T, D), jnp.float32)] * (2 * _NBUF)
            + [pltpu.VMEM((T, D), jnp.float32)]
            + [pltpu.SemaphoreType.DMA] * (2 * _NBUF)
        ),
    )
    def k(x_hbm, w_hbm, o_hbm, *rest):
        ibufs = rest[:_NBUF]
        obufs = rest[_NBUF:2 * _NBUF]
        w_v = rest[2 * _NBUF]
        isems = rest[2 * _NBUF + 1:2 * _NBUF + 1 + _NBUF]
        osems = rest[2 * _NBUF + 1 + _NBUF:]
        wid = lax.axis_index("s") * _NC + lax.axis_index("c")
        base = wid * rows_w

        pltpu.sync_copy(w_hbm, w_v)

        def in_copy(g, s):
            return pltpu.make_async_copy(
                x_hbm.at[pl.ds(base + g * _CB, _CB)], ibufs[s], isems[s])

        def out_copy(g, s):
            return pltpu.make_async_copy(
                obufs[s], o_hbm.at[pl.ds(base + g * _CB, _CB)], osems[s])

        for s in range(_NBUF):
            in_copy(s, s).start()

        def add_chunk(src, dst):
            def t_body(t, c):
                for j in range(D // _L):
                    sl = pl.ds(j * _L, _L)
                    wv = w_v[t, sl]
                    for i in range(_CB):
                        dst[i, t, sl] = src[i, t, sl] + wv
                return c
            lax.fori_loop(0, T, t_body, 0)

        def pair(gg, carry):
            for s in range(_NBUF):
                g = gg * _NBUF + s
                in_copy(g, s).wait()

                @pl.when(g >= _NBUF)
                def _():
                    out_copy(g - _NBUF, s).wait()

                add_chunk(ibufs[s], obufs[s])
                out_copy(g, s).start()

                @pl.when(g + _NBUF < nchunk)
                def _():
                    in_copy(g + _NBUF, s).start()
            return carry

        lax.fori_loop(0, nchunk // _NBUF, pair, 0)

        for s in range(_NBUF):
            out_copy(nchunk - _NBUF + s, s).wait()

    return k


def kernel(x, encoding_weight):
    B, T, D = x.shape
    return _make(B, T, D)(x, encoding_weight)
